# baseline (speedup 1.0000x reference)
import jax
import jax.numpy as jnp
from jax import lax
from jax.experimental import pallas as pl
from jax.experimental.pallas import tpu as pltpu
from jax.experimental.pallas import tpu_sc as plsc

N_NODES = 10000
D_FEAT = 128
N_EDGES = 320000
B_REAL = 2 * N_EDGES
NC, NS, L = 2, 16, 16
NW = NC * NS
CH = 128
CPW = 160
EPW = CPW * CH
B_PAD = NW * EPW
NSLOT = 4
NIDX = 8
NU = 8


def _norms_body(h_ref, n_ref):
    h = h_ref[...]
    n_ref[...] = jnp.sum(h * h, axis=1)


def _node_norms(h):
    return pl.pallas_call(
        _norms_body,
        out_shape=jax.ShapeDtypeStruct((N_NODES,), jnp.float32),
    )(h)


def _sc_scores_body(table, idx_both, norms, out,
                    norms_v, tmp_v,
                    r0, r1, r2, r3, s0, s1, s2, s3,
                    i0, i1, i2, i3, i4, i5, i6, i7,
                    *sems):
    rbuf = [r0, r1, r2, r3]
    sbuf = [s0, s1, s2, s3]
    ibuf = [i0, i1, i2, i3, i4, i5, i6, i7]
    sem_src = sems[0:4]
    sem_add = sems[4:8]
    sem_out = sems[8:12]
    sem_idx = sems[12:20]

    cid = lax.axis_index("c")
    sid = lax.axis_index("s")
    wid = sid * NC + cid
    row0 = wid * CPW
    ebase = wid * EPW

    pltpu.sync_copy(norms, norms_v)

    def compute(c, buf, sb, ib):
        def p1_body(t, carry):
            for u in range(4):
                e = t * 4 + u
                a0 = jnp.zeros((L,), jnp.float32)
                a1 = jnp.zeros((L,), jnp.float32)
                a2 = jnp.zeros((L,), jnp.float32)
                a3 = jnp.zeros((L,), jnp.float32)
                for k in range(D_FEAT // (4 * L)):
                    v0 = buf[e, pl.ds((4 * k) * L, L)]
                    v1 = buf[e, pl.ds((4 * k + 1) * L, L)]
                    v2 = buf[e, pl.ds((4 * k + 2) * L, L)]
                    v3 = buf[e, pl.ds((4 * k + 3) * L, L)]
                    a0 = a0 + v0 * v0
                    a1 = a1 + v1 * v1
                    a2 = a2 + v2 * v2
                    a3 = a3 + v3 * v3
                tmp_v[e, pl.ds(0, L)] = (a0 + a1) + (a2 + a3)
            return carry

        lax.fori_loop(0, CH // 4, p1_body, 0)
        for g in range(CH // L):
            evec = lane + (g * L)
            accs = [jnp.zeros((L,), jnp.float32) for _ in range(4)]
            for k in range(L):
                kvec = jnp.full((L,), k, jnp.int32)
                v = plsc.load_gather(tmp_v, [evec, kvec])
                accs[k % 4] = accs[k % 4] + v
            acc = (accs[0] + accs[1]) + (accs[2] + accs[3])
            si = ib[0, pl.ds(g * L, L)]
            di = ib[1, pl.ds(g * L, L)]
            ns = plsc.load_gather(norms_v, [si])
            nd = plsc.load_gather(norms_v, [di])
            sb[pl.ds(g * L, L)] = 0.5 * acc - 0.5 * ns - 0.5 * nd

    lane = lax.iota(jnp.int32, L)

    def st_idx(c, s):
        pltpu.async_copy(idx_both.at[row0 + c], ibuf[s], sem_idx[s])

    def wt_idx(s):
        pltpu.make_async_copy(idx_both.at[0], ibuf[s], sem_idx[s]).wait()

    def st_src(c, rs, isl):
        pltpu.async_copy(table.at[ibuf[isl].at[0]], rbuf[rs], sem_src[rs])

    def st_add(c, rs, isl):
        pltpu.async_copy(table.at[ibuf[isl].at[1]], rbuf[rs], sem_add[rs],
                         add=True)

    def wt_src(rs):
        pltpu.make_async_copy(table.at[i0.at[0]], rbuf[rs], sem_src[rs]).wait()

    def wt_add(rs):
        pltpu.make_async_copy(table.at[i0.at[0]], rbuf[rs], sem_add[rs]).wait()

    def st_out(c, s):
        pltpu.async_copy(sbuf[s], out.at[pl.ds(ebase + c * CH, CH)], sem_out[s])

    def wt_out(s):
        pltpu.make_async_copy(out.at[pl.ds(0, CH)], sbuf[s], sem_out[s]).wait()

    for c in range(6):
        st_idx(c, c)
    for c in range(3):
        wt_idx(c)
        st_src(c, c, c)
    wt_src(0)
    st_add(0, 0, 0)

    NJ = CPW // NU

    def body(j, carry):
        for u in range(NU):
            c = j * NU + u
            rs = u % NSLOT
            isl = u % NIDX

            if u < 2:
                st_idx(c + 6, (u + 6) % NIDX)
            else:
                @pl.when(j < NJ - 1)
                def _():
                    st_idx(c + 6, (u + 6) % NIDX)

            if u < 5:
                wt_idx((u + 3) % NIDX)
                st_src(c + 3, (u + 3) % NSLOT, (u + 3) % NIDX)
            else:
                @pl.when(j < NJ - 1)
                def _():
                    wt_idx((u + 3) % NIDX)
                    st_src(c + 3, (u + 3) % NSLOT, (u + 3) % NIDX)

            if u < 7:
                wt_src((u + 1) % NSLOT)
                st_add(c + 1, (u + 1) % NSLOT, (u + 1) % NIDX)
            else:
                @pl.when(j < NJ - 1)
                def _():
                    wt_src((u + 1) % NSLOT)
                    st_add(c + 1, (u + 1) % NSLOT, (u + 1) % NIDX)

            wt_add(rs)

            if u < 4:
                @pl.when(j >= 1)
                def _():
                    wt_out(rs)
            else:
                wt_out(rs)

            st_out(c, rs)
        return carry

    lax.fori_loop(0, NJ, body, 0)
    for u in range(NSLOT):
        wt_out(u)


def _sc_scores(table, idx_both, norms):
    mesh = plsc.VectorSubcoreMesh(core_axis_name="c", subcore_axis_name="s")
    return pl.kernel(
        _sc_scores_body,
        out_type=jax.ShapeDtypeStruct((B_PAD,), jnp.float32),
        mesh=mesh,
        compiler_params=pltpu.CompilerParams(needs_layout_passes=False, use_tc_tiling_on_sc=False),
        scratch_types=[
            pltpu.VMEM((N_NODES,), jnp.float32),
            pltpu.VMEM((CH, L + 1), jnp.float32),
            pltpu.VMEM((CH, D_FEAT), jnp.float32),
            pltpu.VMEM((CH, D_FEAT), jnp.float32),
            pltpu.VMEM((CH, D_FEAT), jnp.float32),
            pltpu.VMEM((CH, D_FEAT), jnp.float32),
            pltpu.VMEM((CH,), jnp.float32),
            pltpu.VMEM((CH,), jnp.float32),
            pltpu.VMEM((CH,), jnp.float32),
            pltpu.VMEM((CH,), jnp.float32),
        ] + [pltpu.VMEM((2, CH), jnp.int32)] * 8
          + [pltpu.SemaphoreType.DMA] * 20,
    )(table, idx_both, norms)


def _loss_body(s_ref, o_ref):
    x = s_ref[...]
    r = lax.broadcasted_iota(jnp.int32, x.shape, 0)
    c = lax.broadcasted_iota(jnp.int32, x.shape, 1)
    flat = r * x.shape[1] + c
    y = (flat < N_EDGES).astype(jnp.float32)
    valid = flat < B_REAL
    l = jnp.maximum(x, 0.0) - x * y + jnp.log1p(jnp.exp(-jnp.abs(x)))
    l = jnp.where(valid, l, 0.0)
    o_ref[...] = jnp.reshape(jnp.sum(l) / float(B_REAL), (1, 1))


def _loss(scores):
    out = pl.pallas_call(
        _loss_body,
        out_shape=jax.ShapeDtypeStruct((1, 1), jnp.float32),
    )(scores.reshape(B_PAD // D_FEAT, D_FEAT))
    return out.reshape(())


def kernel(block_outputs, pos_edge_index, neg_edge_index):
    h = block_outputs
    pad = jnp.zeros((B_PAD - B_REAL,), jnp.int32)
    src = jnp.concatenate(
        [pos_edge_index[0].astype(jnp.int32),
         neg_edge_index[0].astype(jnp.int32), pad]).reshape(NW * CPW, CH)
    dst = jnp.concatenate(
        [pos_edge_index[1].astype(jnp.int32),
         neg_edge_index[1].astype(jnp.int32), pad]).reshape(NW * CPW, CH)
    idx_both = jnp.stack([src, dst], axis=1)
    norms = _node_norms(h)
    scores = _sc_scores(h, idx_both, norms)
    return _loss(scores)


# speedup vs baseline: 3.0588x; 3.0588x over previous
import jax
import jax.numpy as jnp
from jax import lax
from jax.experimental import pallas as pl
from jax.experimental.pallas import tpu as pltpu
from jax.experimental.pallas import tpu_sc as plsc

N_NODES = 10000
D_FEAT = 128
N_EDGES = 320000
B_REAL = 2 * N_EDGES
NC, NS, L = 2, 16, 16
NW = NC * NS
CH = 128
CPW = 160
EPW = CPW * CH
B_PAD = NW * EPW
NSLOT = 4
NIDX = 8
NU = 8
GROWS = N_NODES * N_NODES // L

BM = 400


def _gram_body(a_ref, b_ref, o_ref):
    o_ref[...] = jax.lax.dot_general(
        a_ref[...], b_ref[...], (((1,), (1,)), ((), ())),
        preferred_element_type=jnp.float32)


def _gram(h):
    return pl.pallas_call(
        _gram_body,
        grid=(N_NODES // BM,),
        in_specs=[
            pl.BlockSpec((BM, D_FEAT), lambda i: (i, 0)),
            pl.BlockSpec((N_NODES, D_FEAT), lambda i: (0, 0)),
        ],
        out_specs=pl.BlockSpec((BM, N_NODES), lambda i: (i, 0)),
        out_shape=jax.ShapeDtypeStruct((N_NODES, N_NODES), jnp.float32),
    )(h, h)


def _sc_extract_body(g16, idx_both, out,
                     r0, r1, r2, r3, s0, s1, s2, s3,
                     x0, x1, x2, x3,
                     i0, i1, i2, i3, i4, i5, i6, i7,
                     *sems):
    rbuf = [r0, r1, r2, r3]
    sbuf = [s0, s1, s2, s3]
    xbuf = [x0, x1, x2, x3]
    ibuf = [i0, i1, i2, i3, i4, i5, i6, i7]
    sem_gat = sems[0:4]
    sem_out = sems[4:8]
    sem_idx = sems[8:16]

    cid = lax.axis_index("c")
    sid = lax.axis_index("s")
    wid = sid * NC + cid
    row0 = wid * CPW
    ebase = wid * EPW

    lane = lax.iota(jnp.int32, L)

    def st_idx(c, s):
        pltpu.async_copy(idx_both.at[row0 + c], ibuf[s], sem_idx[s])

    def wt_idx(s):
        pltpu.make_async_copy(idx_both.at[0], ibuf[s], sem_idx[s]).wait()

    def rowcalc(isl, rs):
        for g in range(CH // L):
            si = ibuf[isl][0, pl.ds(g * L, L)]
            di = ibuf[isl][1, pl.ds(g * L, L)]
            flat = si * N_NODES + di
            xbuf[rs][pl.ds(g * L, L)] = lax.shift_right_logical(flat, 4)

    def st_gat(rs):
        pltpu.async_copy(g16.at[xbuf[rs]], rbuf[rs], sem_gat[rs])

    def wt_gat(rs):
        pltpu.make_async_copy(g16.at[x0], rbuf[rs], sem_gat[rs]).wait()

    def st_out(c, s):
        pltpu.async_copy(sbuf[s], out.at[pl.ds(ebase + c * CH, CH)], sem_out[s])

    def wt_out(s):
        pltpu.make_async_copy(out.at[pl.ds(0, CH)], sbuf[s], sem_out[s]).wait()

    def compute(gb, sb, isl):
        for g in range(CH // L):
            si = ibuf[isl][0, pl.ds(g * L, L)]
            di = ibuf[isl][1, pl.ds(g * L, L)]
            flat = si * N_NODES + di
            lvec = lax.bitwise_and(flat, jnp.full((L,), L - 1, jnp.int32))
            evec = lane + (g * L)
            sb[pl.ds(g * L, L)] = plsc.load_gather(gb, [evec, lvec])

    for c in range(6):
        st_idx(c, c)
    for c in range(3):
        wt_idx(c)
        rowcalc(c, c)
        st_gat(c)

    NJ = CPW // NU

    def body(j, carry):
        for u in range(NU):
            c = j * NU + u
            rs = u % NSLOT
            isl = u % NIDX

            if u < 2:
                st_idx(c + 6, (u + 6) % NIDX)
            else:
                @pl.when(j < NJ - 1)
                def _():
                    st_idx(c + 6, (u + 6) % NIDX)

            if u < 5:
                wt_idx((u + 3) % NIDX)
                rowcalc((u + 3) % NIDX, (u + 3) % NSLOT)
                st_gat((u + 3) % NSLOT)
            else:
                @pl.when(j < NJ - 1)
                def _():
                    wt_idx((u + 3) % NIDX)
                    rowcalc((u + 3) % NIDX, (u + 3) % NSLOT)
                    st_gat((u + 3) % NSLOT)

            wt_gat(rs)

            if u < 4:
                @pl.when(j >= 1)
                def _():
                    wt_out(rs)
            else:
                wt_out(rs)

            compute(rbuf[rs], sbuf[rs], isl)
            st_out(c, rs)
        return carry

    lax.fori_loop(0, NJ, body, 0)
    for u in range(NSLOT):
        wt_out(u)


def _sc_extract(g16, idx_both):
    mesh = plsc.VectorSubcoreMesh(core_axis_name="c", subcore_axis_name="s")
    return pl.kernel(
        _sc_extract_body,
        out_type=jax.ShapeDtypeStruct((B_PAD,), jnp.float32),
        mesh=mesh,
        compiler_params=pltpu.CompilerParams(
            needs_layout_passes=False, use_tc_tiling_on_sc=False),
        scratch_types=[
            pltpu.VMEM((CH, L), jnp.float32),
            pltpu.VMEM((CH, L), jnp.float32),
            pltpu.VMEM((CH, L), jnp.float32),
            pltpu.VMEM((CH, L), jnp.float32),
            pltpu.VMEM((CH,), jnp.float32),
            pltpu.VMEM((CH,), jnp.float32),
            pltpu.VMEM((CH,), jnp.float32),
            pltpu.VMEM((CH,), jnp.float32),
            pltpu.VMEM((CH,), jnp.int32),
            pltpu.VMEM((CH,), jnp.int32),
            pltpu.VMEM((CH,), jnp.int32),
            pltpu.VMEM((CH,), jnp.int32),
        ] + [pltpu.VMEM((2, CH), jnp.int32)] * 8
          + [pltpu.SemaphoreType.DMA] * 16,
    )(g16, idx_both)


def _loss_body(s_ref, o_ref):
    x = s_ref[...]
    r = lax.broadcasted_iota(jnp.int32, x.shape, 0)
    c = lax.broadcasted_iota(jnp.int32, x.shape, 1)
    flat = r * x.shape[1] + c
    y = (flat < N_EDGES).astype(jnp.float32)
    valid = flat < B_REAL
    l = jnp.maximum(x, 0.0) - x * y + jnp.log1p(jnp.exp(-jnp.abs(x)))
    l = jnp.where(valid, l, 0.0)
    o_ref[...] = jnp.reshape(jnp.sum(l) / float(B_REAL), (1, 1))


def _loss(scores):
    out = pl.pallas_call(
        _loss_body,
        out_shape=jax.ShapeDtypeStruct((1, 1), jnp.float32),
    )(scores.reshape(B_PAD // D_FEAT, D_FEAT))
    return out.reshape(())


def kernel(block_outputs, pos_edge_index, neg_edge_index):
    h = block_outputs
    pad = jnp.zeros((B_PAD - B_REAL,), jnp.int32)
    src = jnp.concatenate(
        [pos_edge_index[0].astype(jnp.int32),
         neg_edge_index[0].astype(jnp.int32), pad]).reshape(NW * CPW, CH)
    dst = jnp.concatenate(
        [pos_edge_index[1].astype(jnp.int32),
         neg_edge_index[1].astype(jnp.int32), pad]).reshape(NW * CPW, CH)
    idx_both = jnp.stack([src, dst], axis=1)
    g = _gram(h)
    g16 = g.reshape(GROWS, L)
    scores = _sc_extract(g16, idx_both)
    return _loss(scores)
